# baseline (speedup 1.0000x reference)
import functools

import jax
import jax.numpy as jnp
from jax import lax
from jax.experimental import pallas as pl
from jax.experimental.pallas import tpu as pltpu
from jax.experimental.pallas import tpu_sc as plsc

B, D, Fd, T = 8, 768, 32, 128
C = 2
D2 = D // C
HW = 196
HWP = 208
NH = HWP // 16
BD = 128
NJ = D // BD
EPS = 1e-8
NEG = -3.0e38


def _pool_copy_body(x_ref, pooled_ref, out_ref):
    xb = x_ref[...]
    pooled_ref[...] = jnp.max(xb, axis=(2, 3)).reshape(1, 1, 1, BD)
    out_ref[...] = xb


def _bcast_body(prev_ref, sel_ref, out_ref):
    s = sel_ref[...]
    out_ref[...] = jnp.broadcast_to(s.reshape(1, BD, 1, 1), (1, BD, Fd, T))


def _splat16(s):
    return jnp.full((16,), s, jnp.float32)


def _perm(x, idx):
    return lax.gather(
        x,
        idx.reshape(16, 1),
        lax.GatherDimensionNumbers(
            offset_dims=(), collapsed_slice_dims=(0,), start_index_map=(0,)),
        (1,),
        mode=lax.GatherScatterMode.PROMISE_IN_BOUNDS,
    )


def _splat_lane(x, lane):
    return _perm(x, jnp.full((16,), lane, jnp.int32))


def _vsum(x):
    lane = lax.iota(jnp.int32, 16)
    for sh in (8, 4, 2, 1):
        x = x + _perm(x, jnp.bitwise_xor(lane, sh))
    return x


def _vmax(x):
    lane = lax.iota(jnp.int32, 16)
    for sh in (8, 4, 2, 1):
        x = jnp.maximum(x, _perm(x, jnp.bitwise_xor(lane, sh)))
    return x


def _vmin_i32(x):
    lane = lax.iota(jnp.int32, 16)
    for sh in (8, 4, 2, 1):
        x = jnp.minimum(x, _perm(x, jnp.bitwise_xor(lane, sh)))
    return x


def _rsqrt16(a):
    i = lax.bitcast_convert_type(a, jnp.int32)
    i = 0x5F3759DF - lax.shift_right_logical(i, 1)
    y = lax.bitcast_convert_type(i, jnp.float32)
    for _ in range(4):
        y = y * (1.5 - 0.5 * a * y * y)
    return y


def _sqrt16(a):
    a = jnp.maximum(a, 1e-35)
    return a * _rsqrt16(a)


def _sc_middle_body(pooled_hbm, v_hbm, sel_hbm, maps_hbm, ml_hbm,
                    v_sp, p_v, maps_v, sel_v, part_v, acc_v, shared):
    core = lax.axis_index("c")
    sub = lax.axis_index("s")
    active = jnp.logical_and(core == 0, sub < B)
    lane = lax.iota(jnp.int32, 16)

    @pl.when(active)
    def _work():
        b = sub
        pltpu.sync_copy(v_hbm.at[b], v_sp)
        pltpu.sync_copy(pooled_hbm.at[b], p_v)

        per_c = []
        for c in range(C):
            c_off = c * D2

            def dloop(d, carry):
                dots, nbs = carry
                base = (d // 16) * 16
                pc = p_v[pl.ds(c_off + base, 16)]
                ps = _splat_lane(pc, d - base)
                ndots = []
                nnbs = []
                for h in range(NH):
                    vv = v_sp[d, pl.ds(h * 16, 16)]
                    ndots.append(dots[h] + ps * vv)
                    nnbs.append(nbs[h] + vv * vv)
                return tuple(ndots), tuple(nnbs)

            zeros = tuple(jnp.zeros((16,), jnp.float32) for _ in range(NH))
            dots, nbs = lax.fori_loop(0, D2, dloop, (zeros, zeros))

            def naloop(k, acc):
                pc = p_v[pl.ds(c_off + k * 16, 16)]
                return acc + pc * pc
            na2 = lax.fori_loop(0, D2 // 16, naloop, jnp.zeros((16,), jnp.float32))
            na = _sqrt16(_vsum(na2))
            dena = jnp.maximum(na, EPS)

            best_val = _splat16(NEG)
            best_idx = jnp.zeros((16,), jnp.int32)
            map_sum = _splat16(0.0)
            for h in range(NH):
                nb = _sqrt16(nbs[h])
                mp = dots[h] / (dena * jnp.maximum(nb, EPS))
                if h == NH - 1:
                    valid = lane < (HW - (NH - 1) * 16)
                    mp = jnp.where(valid, mp, 0.0)
                    mpm = jnp.where(valid, mp, NEG)
                else:
                    mpm = mp
                maps_v[pl.ds(h * 16, 16)] = mp
                map_sum = map_sum + _vsum(mp)
                cmax = _vmax(mpm)
                cidx = _vmin_i32(
                    jnp.where(mpm == cmax, lane + h * 16, jnp.int32(2 ** 30)))
                better = cmax > best_val
                best_idx = jnp.where(better, cidx, best_idx)
                best_val = jnp.where(better, cmax, best_val)

            pltpu.sync_copy(maps_v, maps_hbm.at[b, c])

            for k in range(D2 // 16):
                rows = lane + k * 16
                sv = plsc.load_gather(v_sp, [rows, best_idx])
                sel_v[pl.ds(c_off + k * 16, 16)] = sv

            per_c.append((best_val, map_sum))

        pltpu.sync_copy(sel_v, sel_hbm.at[b])

        def ploop(k, carry):
            d01, n0, n1 = carry
            s0 = sel_v[pl.ds(k * 16, 16)]
            s1 = sel_v[pl.ds(D2 + k * 16, 16)]
            return d01 + s0 * s1, n0 + s0 * s0, n1 + s1 * s1
        z16 = jnp.zeros((16,), jnp.float32)
        d01, n0, n1 = lax.fori_loop(0, D2 // 16, ploop, (z16, z16, z16))
        sn0 = _sqrt16(_vsum(n0))
        sn1 = _sqrt16(_vsum(n1))
        pen = _vsum(d01) / (jnp.maximum(sn0, EPS) * jnp.maximum(sn1, EPS))

        (bv0, ms0), (bv1, ms1) = per_c
        part_v[...] = (-bv0 - bv1) + (ms0 + ms1) * (1.0 / HW) + pen
        pltpu.sync_copy(part_v, shared.at[b])

    plsc.subcore_barrier()

    @pl.when(jnp.logical_and(core == 0, sub == 0))
    def _finalize():
        acc_v[...] = jnp.zeros((16,), jnp.float32)
        for b in range(B):
            pltpu.sync_copy(shared.at[b], part_v)
            acc_v[...] = acc_v[...] + part_v[...]
        acc_v[...] = acc_v[...] * (1.0 / B)
        pltpu.sync_copy(acc_v, ml_hbm)


_sc_middle = functools.partial(
    pl.kernel,
    out_type=[
        jax.ShapeDtypeStruct((B, D), jnp.float32),
        jax.ShapeDtypeStruct((B, C, HWP), jnp.float32),
        jax.ShapeDtypeStruct((16,), jnp.float32),
    ],
    mesh=plsc.VectorSubcoreMesh(core_axis_name="c", subcore_axis_name="s"),
    compiler_params=pltpu.CompilerParams(
        needs_layout_passes=False, use_tc_tiling_on_sc=False),
    scratch_types=[
        pltpu.VMEM((D2, HWP), jnp.float32),
        pltpu.VMEM((D,), jnp.float32),
        pltpu.VMEM((HWP,), jnp.float32),
        pltpu.VMEM((D,), jnp.float32),
        pltpu.VMEM((16,), jnp.float32),
        pltpu.VMEM((16,), jnp.float32),
        pltpu.VMEM_SHARED((B, 16), jnp.float32),
    ],
)(_sc_middle_body)


def _middle_jnp(pooled, v):
    eps = 1e-8
    p = pooled.reshape(B, C, D2)
    na = jnp.sqrt(jnp.sum(p * p, axis=-1))
    nb = jnp.sqrt(jnp.sum(v * v, axis=1))
    dots = jnp.einsum("bcd,bdh->bch", p, v)
    maps = dots / (jnp.maximum(na, eps)[:, :, None] * jnp.maximum(nb, eps)[:, None, :])
    max_ind = jnp.argmax(maps, axis=-1)
    sel = jnp.take_along_axis(v[:, None], max_ind[:, :, None, None], axis=3)[..., 0]
    scores = -jnp.max(maps, axis=-1)
    match_loss = (
        jnp.sum(scores, axis=-1).mean().reshape(1)
        + maps.sum(-1).sum(-1).mean().reshape(1) / HW
    )
    s0, s1 = sel[:, 0], sel[:, 1]
    d01 = jnp.sum(s0 * s1, axis=-1)
    n0 = jnp.sqrt(jnp.sum(s0 * s0, axis=-1))
    n1 = jnp.sqrt(jnp.sum(s1 * s1, axis=-1))
    penalty = (d01 / (jnp.maximum(n0, eps) * jnp.maximum(n1, eps))).mean().reshape(1)
    selflat = sel.reshape(B, D)
    return selflat, maps, match_loss + penalty


def kernel(x, v_ls):
    v = v_ls[0].reshape(B, D2, HW)

    pooled, out1 = pl.pallas_call(
        _pool_copy_body,
        grid=(B, NJ),
        in_specs=[pl.BlockSpec((1, BD, Fd, T), lambda b, j: (b, j, 0, 0))],
        out_specs=[
            pl.BlockSpec((1, 1, 1, BD), lambda b, j: (b, j, 0, 0)),
            pl.BlockSpec((1, BD, Fd, T), lambda b, j: (b, NJ + j, 0, 0)),
        ],
        out_shape=[
            jax.ShapeDtypeStruct((B, NJ, 1, BD), jnp.float32),
            jax.ShapeDtypeStruct((B, 2 * D, Fd, T), jnp.float32),
        ],
    )(x)
    pooled = pooled.reshape(B, D)

    v_pad = jnp.pad(v, ((0, 0), (0, 0), (0, HWP - HW)))
    selflat, maps_p, ml = _sc_middle(pooled, v_pad)
    maps = maps_p[:, :, :HW]
    match_loss = ml[:1]

    out = pl.pallas_call(
        _bcast_body,
        grid=(B, NJ),
        in_specs=[
            pl.BlockSpec((1, 1, 8, 128), lambda b, j: (0, 0, 0, 0)),
            pl.BlockSpec((1, 1, 1, BD), lambda b, j: (b, j, 0, 0)),
        ],
        out_specs=pl.BlockSpec((1, BD, Fd, T), lambda b, j: (b, j, 0, 0)),
        out_shape=jax.ShapeDtypeStruct((B, 2 * D, Fd, T), jnp.float32),
        input_output_aliases={0: 0},
    )(out1, selflat.reshape(B, NJ, 1, BD))

    return out, match_loss, maps.reshape(B, C, 14, 14)
